# baseline (speedup 1.0000x reference)
import functools

import jax
import jax.numpy as jnp
from jax.experimental import pallas as pl
from jax.experimental.pallas import tpu as pltpu


def _router_kernel(x_ref, wr_ref, br_ref, probs_ref, w_ref):
    logits = jax.lax.dot_general(
        x_ref[...], wr_ref[...], (((1,), (1,)), ((), ())),
        preferred_element_type=jnp.float32) + br_ref[...]
    m = jnp.max(logits, axis=-1, keepdims=True)
    ex = jnp.exp(logits - m)
    probs = ex / jnp.sum(ex, axis=-1, keepdims=True)
    probs_ref[...] = probs
    lane = jax.lax.broadcasted_iota(jnp.int32, probs.shape, 1)
    p1 = jnp.max(probs, axis=-1, keepdims=True)
    i1 = jnp.argmax(probs, axis=-1)[:, None]
    masked = jnp.where(lane == i1, -1.0, probs)
    p2 = jnp.max(masked, axis=-1, keepdims=True)
    i2 = jnp.argmax(masked, axis=-1)[:, None]
    w = jnp.where(lane == i1, p1, 0.0) + jnp.where(lane == i2, p2, 0.0)
    w_ref[...] = w / (p1 + p2)


def _moe_dense_kernel(x_ref, w_ref, w1_ref, b1_ref, w2_ref, b2_ref, out_ref):
    e = pl.program_id(1)
    h = pl.program_id(2)

    @pl.when((e == 0) & (h == 0))
    def _init():
        out_ref[...] = jnp.zeros_like(out_ref)

    lane = jax.lax.broadcasted_iota(jnp.int32, w_ref.shape, 1)
    wcol = jnp.sum(jnp.where(lane == e, w_ref[...], 0.0), axis=1, keepdims=True)

    hpre = jax.lax.dot_general(
        x_ref[...], w1_ref[0], (((1,), (1,)), ((), ())),
        preferred_element_type=jnp.float32)
    hact = jnp.maximum(hpre + b1_ref[0], 0.0).astype(jnp.bfloat16)
    part = jax.lax.dot_general(
        hact, w2_ref[0], (((1,), (1,)), ((), ())),
        preferred_element_type=jnp.float32)

    acc = part * wcol

    @pl.when(h == 0)
    def _bias2():
        out_ref[...] += wcol * b2_ref[0]

    out_ref[...] += acc


def kernel(x, Wr, br, W1, b1, W2, b2):
    T, IN = x.shape
    E, H, _ = W1.shape
    OUT = W2.shape[1]

    RB = min(T, 1024)
    probs, w = pl.pallas_call(
        _router_kernel,
        grid=(T // RB,),
        in_specs=[
            pl.BlockSpec((RB, IN), lambda t: (t, 0)),
            pl.BlockSpec((E, IN), lambda t: (0, 0)),
            pl.BlockSpec((1, E), lambda t: (0, 0)),
        ],
        out_specs=[
            pl.BlockSpec((RB, E), lambda t: (t, 0)),
            pl.BlockSpec((RB, E), lambda t: (t, 0)),
        ],
        out_shape=[
            jax.ShapeDtypeStruct((T, E), jnp.float32),
            jax.ShapeDtypeStruct((T, E), jnp.float32),
        ],
    )(x, Wr, br.reshape(1, E))

    TB = min(T, 1024)
    HB = min(H, 512)
    xb = x.astype(jnp.bfloat16)
    W1b = W1.astype(jnp.bfloat16)
    W2b = W2.astype(jnp.bfloat16)

    out = pl.pallas_call(
        _moe_dense_kernel,
        grid=(T // TB, E, H // HB),
        in_specs=[
            pl.BlockSpec((TB, IN), lambda t, e, h: (t, 0)),
            pl.BlockSpec((TB, E), lambda t, e, h: (t, 0)),
            pl.BlockSpec((1, HB, IN), lambda t, e, h: (e, h, 0)),
            pl.BlockSpec((1, 1, HB), lambda t, e, h: (e, 0, h)),
            pl.BlockSpec((1, OUT, HB), lambda t, e, h: (e, 0, h)),
            pl.BlockSpec((1, 1, OUT), lambda t, e, h: (e, 0, 0)),
        ],
        out_specs=pl.BlockSpec((TB, OUT), lambda t, e, h: (t, 0)),
        out_shape=jax.ShapeDtypeStruct((T, OUT), jnp.float32),
        compiler_params=pltpu.CompilerParams(
            dimension_semantics=("parallel", "arbitrary", "arbitrary")),
    )(xb, w, W1b, b1.reshape(E, 1, H), W2b, b2.reshape(E, 1, OUT))

    return out, probs
